# baseline (speedup 1.0000x reference)
import jax
import jax.numpy as jnp
from jax.experimental import pallas as pl
from jax.experimental.pallas import tpu as pltpu

_IGNORE_IOU_THRESH = 0.7
_EPS = 1e-12
_TN = 2048


def _merge_body(bp_ref, gtp_ref, obj_ref, cen_ref, sca_ref, wts_ref, cls_ref,
                obj_o, cen_o, sca_o, wts_o, cls_o, msk_o):
    bp = bp_ref[0]
    gtp = gtp_ref[0]

    x0 = bp[:, 0:1]
    y0 = bp[:, 1:2]
    x1 = bp[:, 2:3]
    y1 = bp[:, 3:4]
    gx0 = gtp[0:1, :]
    gy0 = gtp[1:2, :]
    gx1 = gtp[2:3, :]
    gy1 = gtp[3:4, :]
    area_g = gtp[4:5, :]

    iw = jnp.maximum(jnp.minimum(x1, gx1) - jnp.maximum(x0, gx0), 0.0)
    ih = jnp.maximum(jnp.minimum(y1, gy1) - jnp.maximum(y0, gy0), 0.0)
    inter = iw * ih
    area_p = (x1 - x0) * (y1 - y0)
    iou = inter / (area_p + area_g - inter + _EPS)
    iou_max = jnp.max(iou, axis=1, keepdims=True)
    dyn_obj = jnp.where(iou_max > _IGNORE_IOU_THRESH, -1.0, 0.0)

    obj = obj_ref[0]
    mask = obj > 0.0
    obj_o[0] = jnp.where(mask, obj, dyn_obj)
    cen_o[0] = jnp.where(mask, cen_ref[0], 0.0)
    sca_o[0] = jnp.where(mask, sca_ref[0], 0.0)
    wts_o[0] = jnp.where(mask, wts_ref[0], 0.0)
    cls = cls_ref[0]
    cls_o[0] = jnp.where(mask, cls, -1.0)
    msk_o[0] = jnp.where(mask & (cls >= 0.0), 1.0, 0.0)


@jax.jit
def kernel(box_preds, gt_boxes, obj_t, centers_t, scales_t, weights_t, clas_t):
    b, N, _ = box_preds.shape
    M = gt_boxes.shape[1]
    C = clas_t.shape[-1]

    gx0 = gt_boxes[..., 0]
    gy0 = gt_boxes[..., 1]
    gx1 = gt_boxes[..., 2]
    gy1 = gt_boxes[..., 3]
    area_g = (gx1 - gx0) * (gy1 - gy0)
    gt_pack = jnp.stack([gx0, gy0, gx1, gy1, area_g], axis=1)

    nt = pl.cdiv(N, _TN)
    row_spec = lambda k: pl.BlockSpec((1, _TN, k), lambda i, j: (i, j, 0))

    outs = pl.pallas_call(
        _merge_body,
        grid=(b, nt),
        in_specs=[
            row_spec(4),
            pl.BlockSpec((1, 5, M), lambda i, j: (i, 0, 0)),
            row_spec(1),
            row_spec(2),
            row_spec(2),
            row_spec(2),
            row_spec(C),
        ],
        out_specs=[
            row_spec(1),
            row_spec(2),
            row_spec(2),
            row_spec(2),
            row_spec(C),
            row_spec(C),
        ],
        out_shape=[
            jax.ShapeDtypeStruct((b, N, 1), jnp.float32),
            jax.ShapeDtypeStruct((b, N, 2), jnp.float32),
            jax.ShapeDtypeStruct((b, N, 2), jnp.float32),
            jax.ShapeDtypeStruct((b, N, 2), jnp.float32),
            jax.ShapeDtypeStruct((b, N, C), jnp.float32),
            jax.ShapeDtypeStruct((b, N, C), jnp.float32),
        ],
        compiler_params=pltpu.CompilerParams(
            dimension_semantics=("parallel", "arbitrary"),
        ),
        name="yolov3_target_merge",
    )(box_preds, gt_pack, obj_t, centers_t, scales_t, weights_t, clas_t)
    return tuple(outs)


# speedup vs baseline: 2.2240x; 2.2240x over previous
import jax
import jax.numpy as jnp
from jax.experimental import pallas as pl
from jax.experimental.pallas import tpu as pltpu

_IGNORE_IOU_THRESH = 0.7
_EPS = 1e-12
_TN = 2048


def _merge_body(wt_ref, gt_ref, cls_ref, wout_ref, cls_o, msk_o):
    W = wt_ref[0]
    x0 = W[0:1]
    y0 = W[1:2]
    x1 = W[2:3]
    y1 = W[3:4]
    obj = W[4:5]

    G = gt_ref[0]
    gx0 = G[:, 0:1]
    gy0 = G[:, 1:2]
    gx1 = G[:, 2:3]
    gy1 = G[:, 3:4]
    ga = G[:, 4:5]

    iw = jnp.maximum(jnp.minimum(x1, gx1) - jnp.maximum(x0, gx0), 0.0)
    ih = jnp.maximum(jnp.minimum(y1, gy1) - jnp.maximum(y0, gy0), 0.0)
    inter = iw * ih
    area_p = (x1 - x0) * (y1 - y0)
    iou = inter / ((area_p + ga) - inter + _EPS)
    iou_max = jnp.max(iou, axis=0, keepdims=True)
    dyn = jnp.where(iou_max > _IGNORE_IOU_THRESH, -1.0, 0.0)

    mask = obj > 0.0
    wout_ref[0, 0:1] = jnp.where(mask, obj, dyn)
    wout_ref[0, 1:7] = jnp.where(mask, W[5:11], 0.0)

    maskc = jnp.reshape(obj, (obj.shape[1], 1)) > 0.0
    cls = cls_ref[0]
    cls_o[0] = jnp.where(maskc, cls, -1.0)
    msk_o[0] = jnp.where(maskc & (cls >= 0.0), 1.0, 0.0)


@jax.jit
def kernel(box_preds, gt_boxes, obj_t, centers_t, scales_t, weights_t, clas_t):
    b, N, _ = box_preds.shape
    M = gt_boxes.shape[1]
    C = clas_t.shape[-1]

    wt = jnp.concatenate(
        [box_preds, obj_t, centers_t, scales_t, weights_t], axis=2
    ).transpose(0, 2, 1)

    gx0 = gt_boxes[..., 0]
    gy0 = gt_boxes[..., 1]
    gx1 = gt_boxes[..., 2]
    gy1 = gt_boxes[..., 3]
    area_g = (gx1 - gx0) * (gy1 - gy0)
    gt_pack = jnp.stack([gx0, gy0, gx1, gy1, area_g], axis=-1)

    nt = pl.cdiv(N, _TN)

    wout, cls_o, msk_o = pl.pallas_call(
        _merge_body,
        grid=(b, nt),
        in_specs=[
            pl.BlockSpec((1, 11, _TN), lambda i, j: (i, 0, j)),
            pl.BlockSpec((1, M, 5), lambda i, j: (i, 0, 0)),
            pl.BlockSpec((1, _TN, C), lambda i, j: (i, j, 0)),
        ],
        out_specs=[
            pl.BlockSpec((1, 7, _TN), lambda i, j: (i, 0, j)),
            pl.BlockSpec((1, _TN, C), lambda i, j: (i, j, 0)),
            pl.BlockSpec((1, _TN, C), lambda i, j: (i, j, 0)),
        ],
        out_shape=[
            jax.ShapeDtypeStruct((b, 7, N), jnp.float32),
            jax.ShapeDtypeStruct((b, N, C), jnp.float32),
            jax.ShapeDtypeStruct((b, N, C), jnp.float32),
        ],
        compiler_params=pltpu.CompilerParams(
            dimension_semantics=("parallel", "arbitrary"),
        ),
        name="yolov3_target_merge",
    )(wt, gt_pack, clas_t)

    wo = wout.transpose(0, 2, 1)
    return (
        wo[:, :, 0:1],
        wo[:, :, 1:3],
        wo[:, :, 3:5],
        wo[:, :, 5:7],
        cls_o,
        msk_o,
    )
